# baseline (speedup 1.0000x reference)
import jax
import jax.numpy as jnp
from jax import lax
from jax.experimental import pallas as pl
from jax.experimental.pallas import tpu as pltpu
from jax.experimental.pallas import tpu_sc as plsc

D = 32
HIST = 200
N_ITEMS = 50
GH = 1000
EP_LEN = 200

ITEM_PAD = 64
HIST_PAD = 216
GH_PAD = 1008
ROWS_PER_TILE = 16
N_HTILES = 13

OUT_LEN = N_ITEMS * D + HIST * D + D + EP_LEN
SACH_OFF = N_ITEMS * D
SUC_OFF = SACH_OFF + HIST * D
PRED_OFF = SUC_OFF + D

M_ICB = 0
M_IC = 64
M_HCB = 128
M_HC = 344
M_GH = 560
M_RAT = 1568
M_UCB = 1784
M_UC = 1792
M_CC = 1800
M_NOI = 1808
M_LEN = 2024

NEG_BIG = -1e30


def _body(meta_h, preds_h, utabT_h, rtabT_h, catT_h, out_h,
          meta_v, istrip_v, hstrip_v, ustrip_v, cstrip_v,
          item_v, itemT_v, colstage_v, sach_v, sui_v, suc_v, pred_v,
          item_sh,
          sem_a, sem_h, sem_i, sem_u):
    wid = lax.axis_index("s")

    base = pl.multiple_of(jnp.minimum(wid * ROWS_PER_TILE, HIST), 8)

    pltpu.async_copy(meta_h, meta_v, sem_a).wait()

    iota = lax.iota(jnp.int32, 16)

    hcbw = meta_v[pl.ds(pl.multiple_of(M_HCB + base, 8), 16)]
    h_cps = []
    for hh in range(ROWS_PER_TILE):
        cb = pl.multiple_of(hcbw[hh], 128)
        h_cps.append(pltpu.async_copy(rtabT_h.at[:, pl.ds(cb, 128)],
                                      hstrip_v.at[hh], sem_h))

    @pl.when(wid < 8)
    def _():
        icbw = meta_v[pl.ds(pl.multiple_of(M_ICB + 8 * wid, 8), 16)]
        icw = meta_v[pl.ds(pl.multiple_of(M_IC + 8 * wid, 8), 16)]
        cps = []
        for k in range(8):
            cb = pl.multiple_of(icbw[k], 128)
            cps.append(pltpu.async_copy(rtabT_h.at[:, pl.ds(cb, 128)],
                                        istrip_v.at[k], sem_i))
        for cp in cps:
            cp.wait()
        for k in range(8):
            kv = jnp.full((16,), k, jnp.int32)
            cv = jnp.full((16,), icw[k], jnp.int32)
            colstage_v[pl.ds(0, 16)] = plsc.load_gather(
                istrip_v, [kv, iota, cv])
            colstage_v[pl.ds(16, 16)] = plsc.load_gather(
                istrip_v, [kv, iota + 16, cv])
            pltpu.sync_copy(colstage_v, item_sh.at[8 * wid + k])

    plsc.subcore_barrier()
    pltpu.sync_copy(item_sh, item_v)

    uw = meta_v[pl.ds(M_UCB, 16)]
    g_user = pltpu.async_copy(
        utabT_h.at[:, pl.ds(pl.multiple_of(uw[0], 128), 128)], ustrip_v, sem_u)
    g_cat = pltpu.async_copy(catT_h, cstrip_v, sem_u)

    s1 = jnp.zeros((16,), jnp.float32)
    s2 = jnp.zeros((16,), jnp.float32)
    for b in range(13):
        rf = meta_v[pl.ds(M_RAT + b * 16, 16)].astype(jnp.float32)
        s1 = s1 + rf
        s2 = s2 + rf * rf
    S1 = jnp.sum(s1)
    S2 = jnp.sum(s2)
    r_hist = jnp.float32(1.0 / HIST)
    rmean = S1 * r_hist
    rvar = (S2 - S1 * S1 * r_hist) * jnp.float32(1.0 / (HIST - 1))

    d_lo = iota
    d_hi = iota + 16
    for j in range(ITEM_PAD):
        jv = jnp.full((16,), j, jnp.int32)
        plsc.store_scatter(itemT_v, [d_lo, jv], item_v[j, pl.ds(0, 16)])
        plsc.store_scatter(itemT_v, [d_hi, jv], item_v[j, pl.ds(16, 16)])

    lane_ok = [(iota + 16 * k) < N_ITEMS for k in range(4)]

    for cp in h_cps:
        cp.wait()

    @pl.when(wid < N_HTILES)
    def _():
        ratw = meta_v[pl.ds(pl.multiple_of(M_RAT + base, 8), 16)].astype(
            jnp.float32)
        hcw = meta_v[pl.ds(pl.multiple_of(M_HC + base, 8), 16)]
        noiw = plsc.bitcast(
            meta_v[pl.ds(pl.multiple_of(M_NOI + base, 8), 16)], jnp.float32)
        hcbw2 = meta_v[pl.ds(pl.multiple_of(M_HCB + base, 8), 16)]

        hids = [hcbw2[hh] + hcw[hh] for hh in range(ROWS_PER_TILE)]
        caccs = [jnp.zeros((16,), jnp.float32) for _ in range(ROWS_PER_TILE)]
        for b in range(GH_PAD // 16):
            g = meta_v[pl.ds(M_GH + b * 16, 16)]
            for hh in range(ROWS_PER_TILE):
                caccs[hh] = caccs[hh] + jnp.where(g == hids[hh], 1.0, 0.0)
        cnts = [jnp.sum(caccs[hh]) for hh in range(ROWS_PER_TILE)]

        for g in range(ROWS_PER_TILE // 4):
            rows = [4 * g + r for r in range(4)]
            hrs = []
            for hh in rows:
                mask_s = ((5.0 - ratw[hh]) * 0.2
                          + (rmean * 0.2 + rvar * noiw[hh]) * 0.2)
                m = mask_s * (1.0 - cnts[hh] * 0.1)
                hv = jnp.full((16,), hh, jnp.int32)
                cv = jnp.full((16,), hcw[hh], jnp.int32)
                hrs.append((plsc.load_gather(hstrip_v, [hv, iota, cv]) * m,
                            plsc.load_gather(hstrip_v, [hv, iota + 16, cv])
                            * m))
            ls = [[jnp.zeros((16,), jnp.float32) for _ in range(4)]
                  for _ in range(4)]
            for d in range(D):
                ts = [itemT_v[d, pl.ds(16 * k, 16)] for k in range(4)]
                for r in range(4):
                    sc = hrs[r][0][d] if d < 16 else hrs[r][1][d - 16]
                    for k in range(4):
                        ls[r][k] = ls[r][k] + sc * ts[k]
            es = []
            zs = []
            for r in range(4):
                lr = [jnp.where(lane_ok[k], ls[r][k], NEG_BIG)
                      for k in range(4)]
                mx = jnp.max(jnp.maximum(jnp.maximum(lr[0], lr[1]),
                                         jnp.maximum(lr[2], lr[3])))
                er = [jnp.exp(lr[k] - mx) for k in range(4)]
                es.append(er)
                zs.append(jnp.sum(er[0] + er[1] + er[2] + er[3]))
            a_lo = [jnp.zeros((16,), jnp.float32) for _ in range(4)]
            a_hi = [jnp.zeros((16,), jnp.float32) for _ in range(4)]
            for j in range(N_ITEMS):
                ilo = item_v[j, pl.ds(0, 16)]
                ihi = item_v[j, pl.ds(16, 16)]
                for r in range(4):
                    aj = es[r][j // 16][j % 16]
                    a_lo[r] = a_lo[r] + aj * ilo
                    a_hi[r] = a_hi[r] + aj * ihi
            for r in range(4):
                hh = rows[r]
                sach_v[pl.ds(hh * D, 16)] = a_lo[r] / zs[r]
                sach_v[pl.ds(hh * D + 16, 16)] = a_hi[r] / zs[r]

    @pl.when(wid < N_HTILES - 1)
    def _():
        pltpu.sync_copy(
            sach_v,
            out_h.at[pl.ds(SACH_OFF + wid * (ROWS_PER_TILE * D),
                           ROWS_PER_TILE * D)])

    @pl.when(wid == N_HTILES - 1)
    def _():
        pltpu.sync_copy(
            sach_v.at[pl.ds(0, 8 * D)],
            out_h.at[pl.ds(SACH_OFF + (N_HTILES - 1) * (ROWS_PER_TILE * D),
                           8 * D)])

    g_user.wait()
    g_cat.wait()

    @pl.when(wid == 13)
    def _():
        ucv = jnp.full((16,), uw[8], jnp.int32)
        u_lo = plsc.load_gather(ustrip_v, [iota, ucv])
        u_hi = plsc.load_gather(ustrip_v, [iota + 16, ucv])
        for jj in range(32):
            sui_v[pl.ds(jj * D, 16)] = u_lo * item_v[jj, pl.ds(0, 16)]
            sui_v[pl.ds(jj * D + 16, 16)] = u_hi * item_v[jj, pl.ds(16, 16)]
        pltpu.sync_copy(sui_v, out_h.at[pl.ds(0, 32 * D)])

    @pl.when(wid == 14)
    def _():
        ucv = jnp.full((16,), uw[8], jnp.int32)
        u_lo = plsc.load_gather(ustrip_v, [iota, ucv])
        u_hi = plsc.load_gather(ustrip_v, [iota + 16, ucv])
        for jj in range(18):
            j = 32 + jj
            sui_v[pl.ds(jj * D, 16)] = u_lo * item_v[j, pl.ds(0, 16)]
            sui_v[pl.ds(jj * D + 16, 16)] = u_hi * item_v[j, pl.ds(16, 16)]
        pltpu.sync_copy(sui_v.at[pl.ds(0, 18 * D)],
                        out_h.at[pl.ds(32 * D, 18 * D)])

    @pl.when(wid == 15)
    def _():
        pltpu.sync_copy(preds_h, pred_v)
        pltpu.sync_copy(pred_v, out_h.at[pl.ds(PRED_OFF, EP_LEN)])
        wc = meta_v[pl.ds(M_CC, 16)]
        ucv = jnp.full((16,), uw[8], jnp.int32)
        ccv = jnp.full((16,), wc[0], jnp.int32)
        u_lo = plsc.load_gather(ustrip_v, [iota, ucv])
        u_hi = plsc.load_gather(ustrip_v, [iota + 16, ucv])
        c_lo = plsc.load_gather(cstrip_v, [iota, ccv])
        c_hi = plsc.load_gather(cstrip_v, [iota + 16, ccv])
        suc_v[pl.ds(0, 16)] = u_lo * c_lo
        suc_v[pl.ds(16, 16)] = u_hi * c_hi
        pltpu.sync_copy(suc_v, out_h.at[pl.ds(SUC_OFF, D)])


@jax.jit
def _sc_forward(meta, preds, utabT, rtabT, catT):
    mesh = plsc.VectorSubcoreMesh(core_axis_name="c", subcore_axis_name="s",
                                  num_cores=1)
    f = pl.kernel(
        _body,
        out_type=jax.ShapeDtypeStruct((OUT_LEN,), jnp.float32),
        mesh=mesh,
        compiler_params=pltpu.CompilerParams(needs_layout_passes=False,
                                             use_tc_tiling_on_sc=True),
        scratch_types=[
            pltpu.VMEM((M_LEN,), jnp.int32),
            pltpu.VMEM((8, D, 128), jnp.float32),
            pltpu.VMEM((ROWS_PER_TILE, D, 128), jnp.float32),
            pltpu.VMEM((D, 128), jnp.float32),
            pltpu.VMEM((D, 128), jnp.float32),
            pltpu.VMEM((ITEM_PAD, 128), jnp.float32),
            pltpu.VMEM((D, 128), jnp.float32),
            pltpu.VMEM((128,), jnp.float32),
            pltpu.VMEM((ROWS_PER_TILE * D,), jnp.float32),
            pltpu.VMEM((32 * D,), jnp.float32),
            pltpu.VMEM((D,), jnp.float32),
            pltpu.VMEM((EP_LEN,), jnp.float32),
            pltpu.VMEM_SHARED((ITEM_PAD, 128), jnp.float32),
            pltpu.SemaphoreType.DMA,
            pltpu.SemaphoreType.DMA,
            pltpu.SemaphoreType.DMA,
            pltpu.SemaphoreType.DMA,
        ],
    )
    return f(meta, preds, utabT, rtabT, catT)


def _noise_bits():
    noise = jax.random.normal(jax.random.key(42), (HIST,), dtype=jnp.float32)
    noi = jnp.concatenate([noise, jnp.zeros((HIST_PAD - HIST,), jnp.float32)])
    return jax.lax.bitcast_convert_type(noi, jnp.int32)


def _strip(ids):
    return (ids >> 7) << 7, ids & 127


def kernel(user_ids, item_id, idx, history, global_history, rating, preds,
           last_category, repetition, user_table, recipe_table,
           category_table):
    i32 = jnp.int32
    uid = jnp.asarray(user_ids, i32)
    lc = jnp.asarray(last_category, i32) - 1
    iidx = item_id.astype(i32)
    hidx = history.astype(i32)
    icb, ic = _strip(iidx)
    hcb, hc = _strip(hidx)
    ucb, uc = _strip(uid)
    zpad_i = jnp.zeros((ITEM_PAD - N_ITEMS,), i32)
    zpad_h = jnp.zeros((HIST_PAD - HIST,), i32)
    meta = jnp.concatenate([
        jnp.concatenate([icb, zpad_i]),
        jnp.concatenate([ic, zpad_i]),
        jnp.concatenate([hcb, zpad_h]),
        jnp.concatenate([hc, zpad_h]),
        jnp.concatenate([global_history.astype(i32),
                         jnp.full((GH_PAD - GH,), -1, i32)]),
        jnp.concatenate([rating.astype(i32), zpad_h]),
        jnp.full((8,), ucb, i32),
        jnp.full((8,), uc, i32),
        jnp.full((8,), lc, i32),
        _noise_bits(),
    ])
    catT = jnp.concatenate(
        [category_table, jnp.zeros((128 - 50, D), jnp.float32)]).T
    out = _sc_forward(meta, preds.astype(jnp.float32),
                      user_table.T, recipe_table.T, catT)
    return out.reshape(1, OUT_LEN)


# speedup vs baseline: 1.0008x; 1.0008x over previous
import jax
import jax.numpy as jnp
from jax import lax
from jax.experimental import pallas as pl
from jax.experimental.pallas import tpu as pltpu
from jax.experimental.pallas import tpu_sc as plsc

D = 32
HIST = 200
N_ITEMS = 50
GH = 1000
EP_LEN = 200

ITEM_PAD = 64
HIST_PAD = 216
GH_PAD = 1008
ROWS_PER_TILE = 16
N_HTILES = 13

OUT_LEN = N_ITEMS * D + HIST * D + D + EP_LEN
SACH_OFF = N_ITEMS * D
SUC_OFF = SACH_OFF + HIST * D
PRED_OFF = SUC_OFF + D

M_ICB = 0
M_IC = 64
M_HCB = 128
M_HC = 344
M_GH = 560
M_RAT = 1568
M_UCB = 1784
M_UC = 1792
M_CC = 1800
M_NOI = 1808
M_LEN = 2024

NEG_BIG = -1e30


def _body(meta_h, preds_h, utabT_h, rtabT_h, catT_h, out_h,
          meta_v, istrip_v, hstrip_v, ustrip_v, cstrip_v,
          item_v, itemT_v, colstage_v, sach_v, sui_v, suc_v, pred_v,
          item_sh,
          sem_a, sem_h, sem_i, sem_u):
    wid = lax.axis_index("s")

    base = pl.multiple_of(jnp.minimum(wid * ROWS_PER_TILE, HIST), 8)

    pltpu.async_copy(meta_h, meta_v, sem_a).wait()

    iota = lax.iota(jnp.int32, 16)

    hcbw = meta_v[pl.ds(pl.multiple_of(M_HCB + base, 8), 16)]
    for hh in range(ROWS_PER_TILE):
        cb = pl.multiple_of(hcbw[hh], 128)
        pltpu.async_copy(rtabT_h.at[:, pl.ds(cb, 128)],
                         hstrip_v.at[hh], sem_h)

    @pl.when(wid < 8)
    def _():
        icbw = meta_v[pl.ds(pl.multiple_of(M_ICB + 8 * wid, 8), 16)]
        icw = meta_v[pl.ds(pl.multiple_of(M_IC + 8 * wid, 8), 16)]
        cps = []
        for k in range(8):
            cb = pl.multiple_of(icbw[k], 128)
            cps.append(pltpu.async_copy(rtabT_h.at[:, pl.ds(cb, 128)],
                                        istrip_v.at[k], sem_i))
        for cp in cps:
            cp.wait()
        for k in range(8):
            kv = jnp.full((16,), k, jnp.int32)
            cv = jnp.full((16,), icw[k], jnp.int32)
            colstage_v[pl.ds(0, 16)] = plsc.load_gather(
                istrip_v, [kv, iota, cv])
            colstage_v[pl.ds(16, 16)] = plsc.load_gather(
                istrip_v, [kv, iota + 16, cv])
            pltpu.sync_copy(colstage_v, item_sh.at[8 * wid + k])

    plsc.subcore_barrier()
    pltpu.sync_copy(item_sh, item_v)

    uw = meta_v[pl.ds(M_UCB, 16)]
    g_user = pltpu.async_copy(
        utabT_h.at[:, pl.ds(pl.multiple_of(uw[0], 128), 128)], ustrip_v, sem_u)
    g_cat = pltpu.async_copy(catT_h, cstrip_v, sem_u)

    s1 = jnp.zeros((16,), jnp.float32)
    s2 = jnp.zeros((16,), jnp.float32)
    for b in range(13):
        rf = meta_v[pl.ds(M_RAT + b * 16, 16)].astype(jnp.float32)
        s1 = s1 + rf
        s2 = s2 + rf * rf
    S1 = jnp.sum(s1)
    S2 = jnp.sum(s2)
    r_hist = jnp.float32(1.0 / HIST)
    rmean = S1 * r_hist
    rvar = (S2 - S1 * S1 * r_hist) * jnp.float32(1.0 / (HIST - 1))

    d_lo = iota
    d_hi = iota + 16
    for j in range(ITEM_PAD):
        jv = jnp.full((16,), j, jnp.int32)
        plsc.store_scatter(itemT_v, [d_lo, jv], item_v[j, pl.ds(0, 16)])
        plsc.store_scatter(itemT_v, [d_hi, jv], item_v[j, pl.ds(16, 16)])

    lane_ok = [(iota + 16 * k) < N_ITEMS for k in range(4)]

    @pl.when(wid < N_HTILES)
    def _():
        ratw = meta_v[pl.ds(pl.multiple_of(M_RAT + base, 8), 16)].astype(
            jnp.float32)
        hcw = meta_v[pl.ds(pl.multiple_of(M_HC + base, 8), 16)]
        noiw = plsc.bitcast(
            meta_v[pl.ds(pl.multiple_of(M_NOI + base, 8), 16)], jnp.float32)
        hcbw2 = meta_v[pl.ds(pl.multiple_of(M_HCB + base, 8), 16)]

        hids = [hcbw2[hh] + hcw[hh] for hh in range(ROWS_PER_TILE)]
        caccs = [jnp.zeros((16,), jnp.float32) for _ in range(ROWS_PER_TILE)]
        for b in range(GH_PAD // 16):
            g = meta_v[pl.ds(M_GH + b * 16, 16)]
            for hh in range(ROWS_PER_TILE):
                caccs[hh] = caccs[hh] + jnp.where(g == hids[hh], 1.0, 0.0)
        cnts = [jnp.sum(caccs[hh]) for hh in range(ROWS_PER_TILE)]

        for g in range(ROWS_PER_TILE // 4):
            rows = [4 * g + r for r in range(4)]
            for r in rows:
                pltpu.make_async_copy(rtabT_h.at[:, pl.ds(0, 128)],
                                      hstrip_v.at[r], sem_h).wait()
            hrs = []
            for hh in rows:
                mask_s = ((5.0 - ratw[hh]) * 0.2
                          + (rmean * 0.2 + rvar * noiw[hh]) * 0.2)
                m = mask_s * (1.0 - cnts[hh] * 0.1)
                hv = jnp.full((16,), hh, jnp.int32)
                cv = jnp.full((16,), hcw[hh], jnp.int32)
                hrs.append((plsc.load_gather(hstrip_v, [hv, iota, cv]) * m,
                            plsc.load_gather(hstrip_v, [hv, iota + 16, cv])
                            * m))
            ls = [[jnp.zeros((16,), jnp.float32) for _ in range(4)]
                  for _ in range(4)]
            for d in range(D):
                ts = [itemT_v[d, pl.ds(16 * k, 16)] for k in range(4)]
                for r in range(4):
                    sc = hrs[r][0][d] if d < 16 else hrs[r][1][d - 16]
                    for k in range(4):
                        ls[r][k] = ls[r][k] + sc * ts[k]
            es = []
            zs = []
            for r in range(4):
                lr = [jnp.where(lane_ok[k], ls[r][k], NEG_BIG)
                      for k in range(4)]
                mx = jnp.max(jnp.maximum(jnp.maximum(lr[0], lr[1]),
                                         jnp.maximum(lr[2], lr[3])))
                er = [jnp.exp(lr[k] - mx) for k in range(4)]
                es.append(er)
                zs.append(jnp.sum(er[0] + er[1] + er[2] + er[3]))
            a_lo = [jnp.zeros((16,), jnp.float32) for _ in range(4)]
            a_hi = [jnp.zeros((16,), jnp.float32) for _ in range(4)]
            for j in range(N_ITEMS):
                ilo = item_v[j, pl.ds(0, 16)]
                ihi = item_v[j, pl.ds(16, 16)]
                for r in range(4):
                    aj = es[r][j // 16][j % 16]
                    a_lo[r] = a_lo[r] + aj * ilo
                    a_hi[r] = a_hi[r] + aj * ihi
            for r in range(4):
                hh = rows[r]
                sach_v[pl.ds(hh * D, 16)] = a_lo[r] / zs[r]
                sach_v[pl.ds(hh * D + 16, 16)] = a_hi[r] / zs[r]

    @pl.when(wid < N_HTILES - 1)
    def _():
        pltpu.sync_copy(
            sach_v,
            out_h.at[pl.ds(SACH_OFF + wid * (ROWS_PER_TILE * D),
                           ROWS_PER_TILE * D)])

    @pl.when(wid == N_HTILES - 1)
    def _():
        pltpu.sync_copy(
            sach_v.at[pl.ds(0, 8 * D)],
            out_h.at[pl.ds(SACH_OFF + (N_HTILES - 1) * (ROWS_PER_TILE * D),
                           8 * D)])

    @pl.when(wid >= N_HTILES)
    def _():
        for r in range(ROWS_PER_TILE):
            pltpu.make_async_copy(rtabT_h.at[:, pl.ds(0, 128)],
                                  hstrip_v.at[r], sem_h).wait()

    g_user.wait()
    g_cat.wait()

    @pl.when(wid == 13)
    def _():
        ucv = jnp.full((16,), uw[8], jnp.int32)
        u_lo = plsc.load_gather(ustrip_v, [iota, ucv])
        u_hi = plsc.load_gather(ustrip_v, [iota + 16, ucv])
        for jj in range(32):
            sui_v[pl.ds(jj * D, 16)] = u_lo * item_v[jj, pl.ds(0, 16)]
            sui_v[pl.ds(jj * D + 16, 16)] = u_hi * item_v[jj, pl.ds(16, 16)]
        pltpu.sync_copy(sui_v, out_h.at[pl.ds(0, 32 * D)])

    @pl.when(wid == 14)
    def _():
        ucv = jnp.full((16,), uw[8], jnp.int32)
        u_lo = plsc.load_gather(ustrip_v, [iota, ucv])
        u_hi = plsc.load_gather(ustrip_v, [iota + 16, ucv])
        for jj in range(18):
            j = 32 + jj
            sui_v[pl.ds(jj * D, 16)] = u_lo * item_v[j, pl.ds(0, 16)]
            sui_v[pl.ds(jj * D + 16, 16)] = u_hi * item_v[j, pl.ds(16, 16)]
        pltpu.sync_copy(sui_v.at[pl.ds(0, 18 * D)],
                        out_h.at[pl.ds(32 * D, 18 * D)])

    @pl.when(wid == 15)
    def _():
        pltpu.sync_copy(preds_h, pred_v)
        pltpu.sync_copy(pred_v, out_h.at[pl.ds(PRED_OFF, EP_LEN)])
        wc = meta_v[pl.ds(M_CC, 16)]
        ucv = jnp.full((16,), uw[8], jnp.int32)
        ccv = jnp.full((16,), wc[0], jnp.int32)
        u_lo = plsc.load_gather(ustrip_v, [iota, ucv])
        u_hi = plsc.load_gather(ustrip_v, [iota + 16, ucv])
        c_lo = plsc.load_gather(cstrip_v, [iota, ccv])
        c_hi = plsc.load_gather(cstrip_v, [iota + 16, ccv])
        suc_v[pl.ds(0, 16)] = u_lo * c_lo
        suc_v[pl.ds(16, 16)] = u_hi * c_hi
        pltpu.sync_copy(suc_v, out_h.at[pl.ds(SUC_OFF, D)])


@jax.jit
def _sc_forward(meta, preds, utabT, rtabT, catT):
    mesh = plsc.VectorSubcoreMesh(core_axis_name="c", subcore_axis_name="s",
                                  num_cores=1)
    f = pl.kernel(
        _body,
        out_type=jax.ShapeDtypeStruct((OUT_LEN,), jnp.float32),
        mesh=mesh,
        compiler_params=pltpu.CompilerParams(needs_layout_passes=False,
                                             use_tc_tiling_on_sc=True),
        scratch_types=[
            pltpu.VMEM((M_LEN,), jnp.int32),
            pltpu.VMEM((8, D, 128), jnp.float32),
            pltpu.VMEM((ROWS_PER_TILE, D, 128), jnp.float32),
            pltpu.VMEM((D, 128), jnp.float32),
            pltpu.VMEM((D, 128), jnp.float32),
            pltpu.VMEM((ITEM_PAD, 128), jnp.float32),
            pltpu.VMEM((D, 128), jnp.float32),
            pltpu.VMEM((128,), jnp.float32),
            pltpu.VMEM((ROWS_PER_TILE * D,), jnp.float32),
            pltpu.VMEM((32 * D,), jnp.float32),
            pltpu.VMEM((D,), jnp.float32),
            pltpu.VMEM((EP_LEN,), jnp.float32),
            pltpu.VMEM_SHARED((ITEM_PAD, 128), jnp.float32),
            pltpu.SemaphoreType.DMA,
            pltpu.SemaphoreType.DMA,
            pltpu.SemaphoreType.DMA,
            pltpu.SemaphoreType.DMA,
        ],
    )
    return f(meta, preds, utabT, rtabT, catT)


def _noise_bits():
    noise = jax.random.normal(jax.random.key(42), (HIST,), dtype=jnp.float32)
    noi = jnp.concatenate([noise, jnp.zeros((HIST_PAD - HIST,), jnp.float32)])
    return jax.lax.bitcast_convert_type(noi, jnp.int32)


def _strip(ids):
    return (ids >> 7) << 7, ids & 127


def kernel(user_ids, item_id, idx, history, global_history, rating, preds,
           last_category, repetition, user_table, recipe_table,
           category_table):
    i32 = jnp.int32
    uid = jnp.asarray(user_ids, i32)
    lc = jnp.asarray(last_category, i32) - 1
    iidx = item_id.astype(i32)
    hidx = history.astype(i32)
    icb, ic = _strip(iidx)
    hcb, hc = _strip(hidx)
    ucb, uc = _strip(uid)
    zpad_i = jnp.zeros((ITEM_PAD - N_ITEMS,), i32)
    zpad_h = jnp.zeros((HIST_PAD - HIST,), i32)
    meta = jnp.concatenate([
        jnp.concatenate([icb, zpad_i]),
        jnp.concatenate([ic, zpad_i]),
        jnp.concatenate([hcb, zpad_h]),
        jnp.concatenate([hc, zpad_h]),
        jnp.concatenate([global_history.astype(i32),
                         jnp.full((GH_PAD - GH,), -1, i32)]),
        jnp.concatenate([rating.astype(i32), zpad_h]),
        jnp.full((8,), ucb, i32),
        jnp.full((8,), uc, i32),
        jnp.full((8,), lc, i32),
        _noise_bits(),
    ])
    catT = jnp.concatenate(
        [category_table, jnp.zeros((128 - 50, D), jnp.float32)]).T
    out = _sc_forward(meta, preds.astype(jnp.float32),
                      user_table.T, recipe_table.T, catT)
    return out.reshape(1, OUT_LEN)


# speedup vs baseline: 1.0802x; 1.0794x over previous
import jax
import jax.numpy as jnp
from jax import lax
from jax.experimental import pallas as pl
from jax.experimental.pallas import tpu as pltpu
from jax.experimental.pallas import tpu_sc as plsc

D = 32
HIST = 200
N_ITEMS = 50
GH = 1000
EP_LEN = 200

ITEM_PAD = 64
HIST_PAD = 216
GH_PAD = 1008
ROWS_PER_TILE = 16
N_HTILES = 13

OUT_LEN = N_ITEMS * D + HIST * D + D + EP_LEN
SACH_OFF = N_ITEMS * D
SUC_OFF = SACH_OFF + HIST * D
PRED_OFF = SUC_OFF + D

M_I2 = 0
M_HCB = 144
M_HC = 360
M_GH = 576
M_RAT = 1584
M_UCB = 1800
M_UC = 1808
M_CC = 1816
M_NOI = 1824
M_LEN = 2040

NEG_BIG = -1e30


def _body(meta_h, preds_h, utabT_h, rtabT_h, catT_h, out_h,
          meta_v, istrip_v, hstrip_v, ustrip_v, cstrip_v,
          item_v, itemT_v, colstage_v, sach_v, sui_v, suc_v, pred_v,
          item_sh,
          sem_a, sem_h, sem_i, sem_u, sem_p):
    wid = lax.axis_index("s")

    base = pl.multiple_of(jnp.minimum(wid * ROWS_PER_TILE, HIST), 8)

    pltpu.async_copy(meta_h, meta_v, sem_a).wait()

    iota = lax.iota(jnp.int32, 16)

    hcbw = meta_v[pl.ds(pl.multiple_of(M_HCB + base, 8), 16)]
    for hh in range(ROWS_PER_TILE):
        cb = pl.multiple_of(hcbw[hh], 128)
        pltpu.async_copy(rtabT_h.at[:, pl.ds(cb, 128)],
                         hstrip_v.at[hh], sem_h)

    iw = meta_v[pl.ds(pl.multiple_of(M_I2 + 8 * wid, 8), 16)]
    icps = []
    for k in range(4):
        cb = pl.multiple_of(iw[k], 128)
        icps.append(pltpu.async_copy(rtabT_h.at[:, pl.ds(cb, 128)],
                                     istrip_v.at[k], sem_i))

    uw = meta_v[pl.ds(M_UCB, 16)]
    g_user = pltpu.async_copy(
        utabT_h.at[:, pl.ds(pl.multiple_of(uw[0], 128), 128)], ustrip_v, sem_u)
    g_cat = pltpu.async_copy(catT_h, cstrip_v, sem_u)

    s1 = jnp.zeros((16,), jnp.float32)
    s2 = jnp.zeros((16,), jnp.float32)
    for b in range(13):
        rf = meta_v[pl.ds(M_RAT + b * 16, 16)].astype(jnp.float32)
        s1 = s1 + rf
        s2 = s2 + rf * rf
    S1 = jnp.sum(s1)
    S2 = jnp.sum(s2)
    r_hist = jnp.float32(1.0 / HIST)
    rmean = S1 * r_hist
    rvar = (S2 - S1 * S1 * r_hist) * jnp.float32(1.0 / (HIST - 1))

    ratw = meta_v[pl.ds(pl.multiple_of(M_RAT + base, 8), 16)].astype(
        jnp.float32)
    hcw = meta_v[pl.ds(pl.multiple_of(M_HC + base, 8), 16)]
    noiw = plsc.bitcast(
        meta_v[pl.ds(pl.multiple_of(M_NOI + base, 8), 16)], jnp.float32)
    hcbw2 = meta_v[pl.ds(pl.multiple_of(M_HCB + base, 8), 16)]
    hids = [hcbw2[hh] + hcw[hh] for hh in range(ROWS_PER_TILE)]
    caccs = [jnp.zeros((16,), jnp.float32) for _ in range(ROWS_PER_TILE)]
    for b in range(GH_PAD // 16):
        g = meta_v[pl.ds(M_GH + b * 16, 16)]
        for hh in range(ROWS_PER_TILE):
            caccs[hh] = caccs[hh] + jnp.where(g == hids[hh], 1.0, 0.0)
    cnts = [jnp.sum(caccs[hh]) for hh in range(ROWS_PER_TILE)]

    for cp in icps:
        cp.wait()
    pcps = []
    for k in range(4):
        kv = jnp.full((16,), k, jnp.int32)
        cv = jnp.full((16,), iw[4 + k], jnp.int32)
        colstage_v[k, pl.ds(0, 16)] = plsc.load_gather(
            istrip_v, [kv, iota, cv])
        colstage_v[k, pl.ds(16, 16)] = plsc.load_gather(
            istrip_v, [kv, iota + 16, cv])
        pcps.append(pltpu.async_copy(colstage_v.at[k],
                                     item_sh.at[4 * wid + k], sem_p))
    for cp in pcps:
        cp.wait()
    plsc.subcore_barrier()
    pltpu.sync_copy(item_sh, item_v)

    d_lo = iota
    d_hi = iota + 16
    for j in range(ITEM_PAD):
        jv = jnp.full((16,), j, jnp.int32)
        plsc.store_scatter(itemT_v, [d_lo, jv], item_v[j, pl.ds(0, 16)])
        plsc.store_scatter(itemT_v, [d_hi, jv], item_v[j, pl.ds(16, 16)])

    lane_ok = [(iota + 16 * k) < N_ITEMS for k in range(4)]

    if True:
        for g in range(ROWS_PER_TILE // 4):
            rows = [4 * g + r for r in range(4)]
            for r in rows:
                pltpu.make_async_copy(rtabT_h.at[:, pl.ds(0, 128)],
                                      hstrip_v.at[r], sem_h).wait()
            hrs = []
            for hh in rows:
                mask_s = ((5.0 - ratw[hh]) * 0.2
                          + (rmean * 0.2 + rvar * noiw[hh]) * 0.2)
                m = mask_s * (1.0 - cnts[hh] * 0.1)
                hv = jnp.full((16,), hh, jnp.int32)
                cv = jnp.full((16,), hcw[hh], jnp.int32)
                hrs.append((plsc.load_gather(hstrip_v, [hv, iota, cv]) * m,
                            plsc.load_gather(hstrip_v, [hv, iota + 16, cv])
                            * m))
            ls = [[jnp.zeros((16,), jnp.float32) for _ in range(4)]
                  for _ in range(4)]
            for d in range(D):
                ts = [itemT_v[d, pl.ds(16 * k, 16)] for k in range(4)]
                for r in range(4):
                    sc = hrs[r][0][d] if d < 16 else hrs[r][1][d - 16]
                    for k in range(4):
                        ls[r][k] = ls[r][k] + sc * ts[k]
            es = []
            zs = []
            for r in range(4):
                lr = [jnp.where(lane_ok[k], ls[r][k], NEG_BIG)
                      for k in range(4)]
                mx = jnp.max(jnp.maximum(jnp.maximum(lr[0], lr[1]),
                                         jnp.maximum(lr[2], lr[3])))
                er = [jnp.exp(lr[k] - mx) for k in range(4)]
                es.append(er)
                zs.append(jnp.sum(er[0] + er[1] + er[2] + er[3]))
            a_lo = [jnp.zeros((16,), jnp.float32) for _ in range(4)]
            a_hi = [jnp.zeros((16,), jnp.float32) for _ in range(4)]
            for j in range(N_ITEMS):
                ilo = item_v[j, pl.ds(0, 16)]
                ihi = item_v[j, pl.ds(16, 16)]
                for r in range(4):
                    aj = es[r][j // 16][j % 16]
                    a_lo[r] = a_lo[r] + aj * ilo
                    a_hi[r] = a_hi[r] + aj * ihi
            for r in range(4):
                hh = rows[r]
                sach_v[pl.ds(hh * D, 16)] = a_lo[r] / zs[r]
                sach_v[pl.ds(hh * D + 16, 16)] = a_hi[r] / zs[r]

    @pl.when(wid < N_HTILES - 1)
    def _():
        pltpu.sync_copy(
            sach_v,
            out_h.at[pl.ds(SACH_OFF + wid * (ROWS_PER_TILE * D),
                           ROWS_PER_TILE * D)])

    @pl.when(wid == N_HTILES - 1)
    def _():
        pltpu.sync_copy(
            sach_v.at[pl.ds(0, 8 * D)],
            out_h.at[pl.ds(SACH_OFF + (N_HTILES - 1) * (ROWS_PER_TILE * D),
                           8 * D)])

    g_user.wait()
    g_cat.wait()

    @pl.when(wid == 13)
    def _():
        ucv = jnp.full((16,), uw[8], jnp.int32)
        u_lo = plsc.load_gather(ustrip_v, [iota, ucv])
        u_hi = plsc.load_gather(ustrip_v, [iota + 16, ucv])
        for jj in range(32):
            sui_v[pl.ds(jj * D, 16)] = u_lo * item_v[jj, pl.ds(0, 16)]
            sui_v[pl.ds(jj * D + 16, 16)] = u_hi * item_v[jj, pl.ds(16, 16)]
        pltpu.sync_copy(sui_v, out_h.at[pl.ds(0, 32 * D)])

    @pl.when(wid == 14)
    def _():
        ucv = jnp.full((16,), uw[8], jnp.int32)
        u_lo = plsc.load_gather(ustrip_v, [iota, ucv])
        u_hi = plsc.load_gather(ustrip_v, [iota + 16, ucv])
        for jj in range(18):
            j = 32 + jj
            sui_v[pl.ds(jj * D, 16)] = u_lo * item_v[j, pl.ds(0, 16)]
            sui_v[pl.ds(jj * D + 16, 16)] = u_hi * item_v[j, pl.ds(16, 16)]
        pltpu.sync_copy(sui_v.at[pl.ds(0, 18 * D)],
                        out_h.at[pl.ds(32 * D, 18 * D)])

    @pl.when(wid == 15)
    def _():
        pltpu.sync_copy(preds_h, pred_v)
        pltpu.sync_copy(pred_v, out_h.at[pl.ds(PRED_OFF, EP_LEN)])
        wc = meta_v[pl.ds(M_CC, 16)]
        ucv = jnp.full((16,), uw[8], jnp.int32)
        ccv = jnp.full((16,), wc[0], jnp.int32)
        u_lo = plsc.load_gather(ustrip_v, [iota, ucv])
        u_hi = plsc.load_gather(ustrip_v, [iota + 16, ucv])
        c_lo = plsc.load_gather(cstrip_v, [iota, ccv])
        c_hi = plsc.load_gather(cstrip_v, [iota + 16, ccv])
        suc_v[pl.ds(0, 16)] = u_lo * c_lo
        suc_v[pl.ds(16, 16)] = u_hi * c_hi
        pltpu.sync_copy(suc_v, out_h.at[pl.ds(SUC_OFF, D)])


@jax.jit
def _sc_forward(meta, preds, utabT, rtabT, catT):
    mesh = plsc.VectorSubcoreMesh(core_axis_name="c", subcore_axis_name="s",
                                  num_cores=1)
    f = pl.kernel(
        _body,
        out_type=jax.ShapeDtypeStruct((OUT_LEN,), jnp.float32),
        mesh=mesh,
        compiler_params=pltpu.CompilerParams(needs_layout_passes=False,
                                             use_tc_tiling_on_sc=True),
        scratch_types=[
            pltpu.VMEM((M_LEN,), jnp.int32),
            pltpu.VMEM((4, D, 128), jnp.float32),
            pltpu.VMEM((ROWS_PER_TILE, D, 128), jnp.float32),
            pltpu.VMEM((D, 128), jnp.float32),
            pltpu.VMEM((D, 128), jnp.float32),
            pltpu.VMEM((ITEM_PAD, 128), jnp.float32),
            pltpu.VMEM((D, 128), jnp.float32),
            pltpu.VMEM((4, 128), jnp.float32),
            pltpu.VMEM((ROWS_PER_TILE * D,), jnp.float32),
            pltpu.VMEM((32 * D,), jnp.float32),
            pltpu.VMEM((D,), jnp.float32),
            pltpu.VMEM((EP_LEN,), jnp.float32),
            pltpu.VMEM_SHARED((ITEM_PAD, 128), jnp.float32),
            pltpu.SemaphoreType.DMA,
            pltpu.SemaphoreType.DMA,
            pltpu.SemaphoreType.DMA,
            pltpu.SemaphoreType.DMA,
            pltpu.SemaphoreType.DMA,
        ],
    )
    return f(meta, preds, utabT, rtabT, catT)


def _noise_bits():
    noise = jax.random.normal(jax.random.key(42), (HIST,), dtype=jnp.float32)
    noi = jnp.concatenate([noise, jnp.zeros((HIST_PAD - HIST,), jnp.float32)])
    return jax.lax.bitcast_convert_type(noi, jnp.int32)


def _strip(ids):
    return (ids >> 7) << 7, ids & 127


def kernel(user_ids, item_id, idx, history, global_history, rating, preds,
           last_category, repetition, user_table, recipe_table,
           category_table):
    i32 = jnp.int32
    uid = jnp.asarray(user_ids, i32)
    lc = jnp.asarray(last_category, i32) - 1
    iidx = item_id.astype(i32)
    hidx = history.astype(i32)
    icb, ic = _strip(iidx)
    hcb, hc = _strip(hidx)
    ucb, uc = _strip(uid)
    zpad_i = jnp.zeros((ITEM_PAD - N_ITEMS,), i32)
    zpad_h = jnp.zeros((HIST_PAD - HIST,), i32)
    icb64 = jnp.concatenate([icb, zpad_i])
    ic64 = jnp.concatenate([ic, zpad_i])
    i2 = jnp.concatenate([icb64.reshape(16, 4), ic64.reshape(16, 4)],
                         axis=1).reshape(128)
    meta = jnp.concatenate([
        jnp.concatenate([i2, jnp.zeros((16,), i32)]),
        jnp.concatenate([hcb, zpad_h]),
        jnp.concatenate([hc, zpad_h]),
        jnp.concatenate([global_history.astype(i32),
                         jnp.full((GH_PAD - GH,), -1, i32)]),
        jnp.concatenate([rating.astype(i32), zpad_h]),
        jnp.full((8,), ucb, i32),
        jnp.full((8,), uc, i32),
        jnp.full((8,), lc, i32),
        _noise_bits(),
    ])
    catT = jnp.concatenate(
        [category_table, jnp.zeros((128 - 50, D), jnp.float32)]).T
    out = _sc_forward(meta, preds.astype(jnp.float32),
                      user_table.T, recipe_table.T, catT)
    return out.reshape(1, OUT_LEN)
